# Initial kernel scaffold; baseline (speedup 1.0000x reference)
import jax
import jax.numpy as jnp
from jax.experimental import pallas as pl


def kernel(x, embed):
    raise NotImplementedError("write your pallas kernel here")


# speedup vs baseline: 1.4016x; 1.4016x over previous
import functools

import jax
import jax.numpy as jnp
from jax.experimental import pallas as pl

DIM = 64
CODEBOOK_SIZE = 1024
ROWS_PER_BLOCK = 2304


def _tc_body(x_ref, e_ref, idx_ref, q_ref):
    x = x_ref[...]
    e = e_ref[...]
    cross = jax.lax.dot_general(
        x, e, (((1,), (1,)), ((), ())),
        preferred_element_type=jnp.float32,
    )
    x_sq = jnp.sum(x * x, axis=1, keepdims=True)
    e_sq = jnp.sum(e * e, axis=1)[None, :]
    sq = jnp.maximum(x_sq - 2.0 * cross + e_sq, 0.0)
    dist = jnp.sqrt(sq)
    idx = jnp.argmin(dist, axis=1).astype(jnp.int32)
    idx_ref[0, 0, :] = idx
    onehot = (jax.lax.broadcasted_iota(jnp.int32, cross.shape, 1)
              == idx[:, None]).astype(jnp.float32)
    q_ref[...] = jax.lax.dot_general(
        onehot, e, (((1,), (0,)), ((), ())),
        preferred_element_type=jnp.float32,
        precision=jax.lax.Precision.HIGHEST,
    )


@jax.jit
def kernel(x, embed):
    b, n, d = x.shape
    m = b * n
    nblk = m // ROWS_PER_BLOCK
    xf = x.reshape(m, d).astype(jnp.float32)
    e2d = embed[0].astype(jnp.float32)

    idx3, quant = pl.pallas_call(
        _tc_body,
        grid=(nblk,),
        in_specs=[
            pl.BlockSpec((ROWS_PER_BLOCK, d), lambda i: (i, 0)),
            pl.BlockSpec((CODEBOOK_SIZE, d), lambda i: (0, 0)),
        ],
        out_specs=[
            pl.BlockSpec((1, 1, ROWS_PER_BLOCK), lambda i: (i, 0, 0)),
            pl.BlockSpec((ROWS_PER_BLOCK, d), lambda i: (i, 0)),
        ],
        out_shape=[
            jax.ShapeDtypeStruct((nblk, 1, ROWS_PER_BLOCK), jnp.int32),
            jax.ShapeDtypeStruct((m, d), jnp.float32),
        ],
    )(xf, e2d)

    embed_ind = idx3.reshape(b, n)
    quantize = quant.reshape(b, n, d)
    return quantize, embed_ind


# speedup vs baseline: 2.1300x; 1.5197x over previous
import functools

import jax
import jax.numpy as jnp
from jax import lax
from jax.experimental import pallas as pl
from jax.experimental.pallas import tpu as pltpu
from jax.experimental.pallas import tpu_sc as plsc

DIM = 64
CODEBOOK_SIZE = 1024
ROWS_PER_BLOCK = 2304
TOTAL_ROWS = 36864

_INFO = plsc.get_sparse_core_info()
_NC, _NS = _INFO.num_cores, _INFO.num_subcores
_NW = _NC * _NS
_BPW = TOTAL_ROWS // _NW
_CHUNK = 128
_NCHUNK = _BPW // _CHUNK


def _tc_body(x_ref, e_ref, idx_ref):
    x = x_ref[...]
    e = e_ref[...]
    cross = jax.lax.dot_general(
        x, e, (((1,), (1,)), ((), ())),
        preferred_element_type=jnp.float32,
    )
    x_sq = jnp.sum(x * x, axis=1, keepdims=True)
    e_sq = jnp.sum(e * e, axis=1)[None, :]
    sq = jnp.maximum(x_sq - 2.0 * cross + e_sq, 0.0)
    dist = jnp.sqrt(sq)
    idx = jnp.argmin(dist, axis=1).astype(jnp.int32)
    idx_ref[0, 0, :] = idx


def _sc_gather_body(table_hbm, idx_hbm, out_hbm, idx_v, rows_v, sem):
    wid = lax.axis_index("s") * _NC + lax.axis_index("c")
    pltpu.sync_copy(idx_hbm.at[wid], idx_v)
    copies = []
    for j in range(_NCHUNK):
        copies.append(pltpu.async_copy(
            table_hbm.at[idx_v.at[j]],
            rows_v.at[pl.ds(j * _CHUNK, _CHUNK)],
            sem,
        ))
    for c in copies:
        c.wait()
    pltpu.sync_copy(rows_v, out_hbm.at[pl.ds(wid * _BPW, _BPW)])


@functools.partial(
    pl.kernel,
    mesh=plsc.VectorSubcoreMesh(core_axis_name="c", subcore_axis_name="s"),
    out_type=jax.ShapeDtypeStruct((TOTAL_ROWS, DIM), jnp.float32),
    scratch_types=[
        pltpu.VMEM((_NCHUNK, _CHUNK), jnp.int32),
        pltpu.VMEM((_BPW, DIM), jnp.float32),
        pltpu.SemaphoreType.DMA,
    ],
    compiler_params=pltpu.CompilerParams(use_tc_tiling_on_sc=False),
)
def _sc_gather(table_hbm, idx_hbm, out_hbm, idx_v, rows_v, sem):
    _sc_gather_body(table_hbm, idx_hbm, out_hbm, idx_v, rows_v, sem)


@jax.jit
def kernel(x, embed):
    b, n, d = x.shape
    m = b * n
    nblk = m // ROWS_PER_BLOCK
    xf = x.reshape(m, d).astype(jnp.float32)
    e2d = embed[0].astype(jnp.float32)

    idx3 = pl.pallas_call(
        _tc_body,
        grid=(nblk,),
        in_specs=[
            pl.BlockSpec((ROWS_PER_BLOCK, d), lambda i: (i, 0)),
            pl.BlockSpec((CODEBOOK_SIZE, d), lambda i: (0, 0)),
        ],
        out_specs=pl.BlockSpec((1, 1, ROWS_PER_BLOCK), lambda i: (i, 0, 0)),
        out_shape=jax.ShapeDtypeStruct((nblk, 1, ROWS_PER_BLOCK), jnp.int32),
    )(xf, e2d)

    idx_flat = idx3.reshape(m)
    quant = _sc_gather(e2d, idx_flat.reshape(_NW, _NCHUNK, _CHUNK))
    embed_ind = idx_flat.reshape(b, n)
    quantize = quant.reshape(b, n, d)
    return quantize, embed_ind
